# baseline (speedup 1.0000x reference)
import functools

import jax
import jax.numpy as jnp
from jax import lax
from jax.experimental import pallas as pl
from jax.experimental.pallas import tpu as pltpu
from jax.experimental.pallas import tpu_sc as plsc

N = 10000
D = 128
HD = 64
E = 320000
OUT = 32

NSUB = 16
CHUNK = 128
NCH = 160
SUP = 1
EPAD = NSUB * NCH * CHUNK
NACC = 10112
RPT = NACC // NSUB

_mesh = plsc.VectorSubcoreMesh(core_axis_name="c", subcore_axis_name="s")


NSC = NCH // SUP


def _sc_agg(do_cnt, hu2, hi2, s2eu, s2ou, dstu, s2ei, s2oi, dsti,
            zeros, zeros16, ones16):
    out_type = [jax.ShapeDtypeStruct((NACC, HD), jnp.float32)
                for _ in range(4)]
    scratch = [pltpu.VMEM((NSC, SUP * CHUNK), jnp.int32),
               pltpu.VMEM((NSC, SUP * CHUNK), jnp.int32),
               pltpu.VMEM((SUP * CHUNK, HD), jnp.float32),
               pltpu.VMEM((SUP * CHUNK, HD), jnp.float32),
               pltpu.VMEM((SUP * CHUNK, HD), jnp.float32),
               pltpu.VMEM((SUP * CHUNK, HD), jnp.float32),
               pltpu.VMEM_SHARED((NACC, HD), jnp.float32),
               pltpu.VMEM((SUP * CHUNK, 16), jnp.float32),
               pltpu.SemaphoreType.DMA, pltpu.SemaphoreType.DMA,
               pltpu.SemaphoreType.DMA, pltpu.SemaphoreType.DMA,
               pltpu.SemaphoreType.DMA, pltpu.SemaphoreType.DMA,
               pltpu.SemaphoreType.DMA, pltpu.SemaphoreType.DMA]
    if do_cnt:
        out_type += [jax.ShapeDtypeStruct((NACC, 16), jnp.float32),
                     jax.ShapeDtypeStruct((NACC, 16), jnp.float32)]
        scratch += [pltpu.VMEM_SHARED((NACC, 16), jnp.float32)]

    @functools.partial(pl.kernel, out_type=out_type, mesh=_mesh,
                       scratch_types=scratch,
                       compiler_params=pltpu.CompilerParams(
                           use_tc_tiling_on_sc=False))
    def run(hu2, hi2, s2eu, s2ou, dstu, s2ei, s2oi, dsti,
            zeros, zeros16, ones16, *rest):
        if do_cnt:
            (Si0, Si1, Su0, Su1, cnt_i, cnt_u, idxb, idxd,
             gb0, gb1, gb2, gb3, acc, ones_v,
             sg0, sg1, sg2, sg3, ss0, ss1, ss2, ss3, cacc) = rest
        else:
            (Si0, Si1, Su0, Su1, idxb, idxd,
             gb0, gb1, gb2, gb3, acc, ones_v,
             sg0, sg1, sg2, sg3, ss0, ss1, ss2, ss3) = rest
            cnt_i = cnt_u = cacc = None
        bufs = ((gb0, sg0, ss0), (gb1, sg1, ss1),
                (gb2, sg2, ss2), (gb3, sg3, ss3))
        s = lax.axis_index("s")
        c = lax.axis_index("c")

        def body(h_hbm, s2e_hbm, s2o_hbm, dst_hbm, out0, out1, cnt_hbm):
            pltpu.sync_copy(dst_hbm.at[pl.ds(s * NSC, NSC)], idxd)
            if do_cnt:
                pltpu.sync_copy(ones16, ones_v)

            for p, (src_hbm, out_p) in ((0, (s2e_hbm, out0)),
                                        (1, (s2o_hbm, out1))):
                pltpu.sync_copy(zeros.at[pl.ds(s * RPT, RPT)],
                                acc.at[pl.ds(s * RPT, RPT)])
                if do_cnt and p == 0:
                    pltpu.sync_copy(zeros16.at[pl.ds(s * RPT, RPT)],
                                    cacc.at[pl.ds(s * RPT, RPT)])
                pltpu.sync_copy(src_hbm.at[pl.ds(s * NSC, NSC)], idxb)
                plsc.subcore_barrier()

                for b in range(2):
                    gb, sg, ss = bufs[b]
                    pltpu.async_copy(h_hbm.at[idxb.at[b]], gb, sg)

                @pl.loop(0, NSC, step=4)
                def _(t):
                    for b in range(4):
                        tt = t + b
                        gb, sg, ss = bufs[b]
                        gb2, sg2, ss2 = bufs[(b + 2) % 4]
                        pltpu.make_async_copy(h_hbm.at[idxb.at[tt]],
                                              gb, sg).wait()
                        pltpu.async_copy(gb, acc.at[idxd.at[tt]],
                                         ss, add=True)
                        if do_cnt and p == 0:
                            pltpu.sync_copy(ones_v, cacc.at[idxd.at[tt]],
                                            add=True)

                        @pl.when(tt >= 2)
                        def _():
                            pltpu.make_async_copy(
                                gb2, acc.at[idxd.at[tt - 2]], ss2).wait()

                        @pl.when(tt + 2 < NSC)
                        def _():
                            pltpu.async_copy(h_hbm.at[idxb.at[tt + 2]],
                                             gb2, sg2)

                for tt in (NSC - 2, NSC - 1):
                    gb, sg, ss = bufs[tt % 4]
                    pltpu.make_async_copy(gb, acc.at[idxd.at[tt]],
                                          ss).wait()

                plsc.subcore_barrier()
                pltpu.sync_copy(acc.at[pl.ds(s * RPT, RPT)],
                                out_p.at[pl.ds(s * RPT, RPT)])
                if do_cnt and p == 0:
                    pltpu.sync_copy(cacc.at[pl.ds(s * RPT, RPT)],
                                    cnt_hbm.at[pl.ds(s * RPT, RPT)])

        @pl.when(c == 0)
        def _():
            body(hu2, s2eu, s2ou, dstu, Si0, Si1, cnt_i)

        @pl.when(c == 1)
        def _():
            body(hi2, s2ei, s2oi, dsti, Su0, Su1, cnt_u)

    return run(hu2, hi2, s2eu, s2ou, dstu, s2ei, s2oi, dsti,
               zeros, zeros16, ones16)


def _tc_layer(final, Su0, Su1, Si0, Si1, hu, hi, cu, ci,
              Wl_u2i, bl_u2i, Wr_u2i, Wl_i2u, bl_i2u, Wr_i2u,
              Wlin_u=None, blin_u=None, Wlin_i=None, blin_i=None):
    odim = OUT if final else D

    def body(Su0_r, Su1_r, Si0_r, Si1_r, hu_r, hi_r, cu_r, ci_r,
             wlu, blu, wru, wli, bli, wri, *rest):
        inv_i = 1.0 / jnp.maximum(ci_r[:, 0:1], 1.0)
        inv_u = 1.0 / jnp.maximum(cu_r[:, 0:1], 1.0)
        agg_i = ((Si0_r[...] * inv_i) @ wlu[0:HD, :]
                 + (Si1_r[...] * inv_i) @ wlu[HD:D, :])
        agg_u = ((Su0_r[...] * inv_u) @ wli[0:HD, :]
                 + (Su1_r[...] * inv_u) @ wli[HD:D, :])
        ni = jnp.maximum(agg_i + blu[...] + hi_r[...] @ wru[...], 0.0)
        nu = jnp.maximum(agg_u + bli[...] + hu_r[...] @ wri[...], 0.0)
        if final:
            wlinu, blinu, wlini, blini, out_u, out_i = rest
            out_u[...] = nu @ wlinu[...] + blinu[...]
            out_i[...] = ni @ wlini[...] + blini[...]
        else:
            out_u, out_i = rest
            out_u[...] = nu
            out_i[...] = ni

    args = [Su0, Su1, Si0, Si1, hu, hi, cu, ci,
            Wl_u2i, bl_u2i, Wr_u2i, Wl_i2u, bl_i2u, Wr_i2u]
    if final:
        args += [Wlin_u, blin_u, Wlin_i, blin_i]
    BRW = 1000
    blocked = lambda cols: pl.BlockSpec((BRW, cols), lambda i: (i, 0))
    full = lambda a: pl.BlockSpec(a.shape, lambda i: (0, 0))
    in_specs = [blocked(HD)] * 4 + [blocked(D)] * 2 + [blocked(16)] * 2
    in_specs += [full(a) for a in args[8:]]
    return pl.pallas_call(
        body,
        grid=(N // BRW,),
        in_specs=in_specs,
        out_specs=[blocked(odim), blocked(odim)],
        out_shape=[jax.ShapeDtypeStruct((N, odim), jnp.float32),
                   jax.ShapeDtypeStruct((N, odim), jnp.float32)],
    )(*args)


def _tc_decoder(z_u, z_i):
    BR = 200

    def body(zu_r, zi_r, out_r):
        logits = lax.dot_general(zu_r[...], zi_r[...],
                                 (((1,), (1,)), ((), ())),
                                 preferred_element_type=jnp.float32)
        out_r[...] = 1.0 / (1.0 + jnp.exp(-logits))

    return pl.pallas_call(
        body,
        grid=(N // BR,),
        in_specs=[pl.BlockSpec((BR, OUT), lambda i: (i, 0)),
                  pl.BlockSpec((N, OUT), lambda i: (0, 0))],
        out_specs=pl.BlockSpec((BR, N), lambda i: (i, 0)),
        out_shape=jax.ShapeDtypeStruct((N, N), jnp.float32),
    )(z_u, z_i)


def _prep_edges(ei):
    src = ei[0].astype(jnp.int32)
    dst = ei[1].astype(jnp.int32)
    pad = EPAD - E
    src = jnp.concatenate([src, jnp.zeros((pad,), jnp.int32)])
    dst = jnp.concatenate([dst, jnp.full((pad,), N, jnp.int32)])
    s2 = src + src
    shape = (NSUB * NSC, SUP * CHUNK)
    return s2.reshape(shape), (s2 + 1).reshape(shape), dst.reshape(shape)


def kernel(x_user, x_item, params, edge_index_u2i, edge_index_i2u):
    s2eu, s2ou, dstu = _prep_edges(edge_index_u2i)
    s2ei, s2oi, dsti = _prep_edges(edge_index_i2u)
    zeros = jnp.zeros((NACC, HD), jnp.float32)
    zeros16 = jnp.zeros((NACC, 16), jnp.float32)
    ones16 = jnp.ones((SUP * CHUNK, 16), jnp.float32)

    p = params
    b2 = lambda v: v.reshape(1, -1)

    hu, hi = x_user, x_item
    cu = ci = None
    for L in range(3):
        hu2 = hu.reshape(2 * N, HD)
        hi2 = hi.reshape(2 * N, HD)
        res = _sc_agg(L == 0, hu2, hi2, s2eu, s2ou, dstu, s2ei, s2oi, dsti,
                      zeros, zeros16, ones16)
        if L == 0:
            Si0, Si1, Su0, Su1, ci_f, cu_f = res
            ci = ci_f[:N]
            cu = cu_f[:N]
        else:
            Si0, Si1, Su0, Su1 = res
        final = L == 2
        extra = {}
        if final:
            extra = dict(Wlin_u=p['Wlin_user'], blin_u=b2(p['blin_user']),
                         Wlin_i=p['Wlin_item'], blin_i=b2(p['blin_item']))
        hu, hi = _tc_layer(final, Su0[:N], Su1[:N], Si0[:N], Si1[:N],
                           hu, hi, cu, ci,
                           p['Wl%d_u2i' % L], b2(p['bl%d_u2i' % L]),
                           p['Wr%d_u2i' % L],
                           p['Wl%d_i2u' % L], b2(p['bl%d_i2u' % L]),
                           p['Wr%d_i2u' % L], **extra)
    return _tc_decoder(hu, hi)


# speedup vs baseline: 1.7616x; 1.7616x over previous
import functools

import jax
import jax.numpy as jnp
from jax import lax
from jax.experimental import pallas as pl
from jax.experimental.pallas import tpu as pltpu
from jax.experimental.pallas import tpu_sc as plsc

N = 10000
D = 128
E = 320000
OUT = 32

NSUB = 16
CHUNK = 256
NCH = 80
EPAD = NSUB * NCH * CHUNK
NACC = 10112
RPT = NACC // NSUB

_mesh = plsc.VectorSubcoreMesh(core_axis_name="c", subcore_axis_name="s")


def _sc_agg(do_cnt, hub, hib, srcu, dstu, srci, dsti, zeros, zeros16, ones16):
    out_type = [jax.ShapeDtypeStruct((NACC, D), jnp.bfloat16)
                for _ in range(2)]
    scratch = [pltpu.VMEM((NCH, CHUNK), jnp.int32),
               pltpu.VMEM((NCH, CHUNK), jnp.int32),
               pltpu.VMEM((CHUNK, D), jnp.bfloat16),
               pltpu.VMEM((CHUNK, D), jnp.bfloat16),
               pltpu.VMEM_SHARED((NACC, D), jnp.bfloat16),
               pltpu.VMEM((CHUNK, 16), jnp.float32),
               pltpu.SemaphoreType.DMA, pltpu.SemaphoreType.DMA,
               pltpu.SemaphoreType.DMA, pltpu.SemaphoreType.DMA]
    if do_cnt:
        out_type += [jax.ShapeDtypeStruct((NACC, 16), jnp.float32),
                     jax.ShapeDtypeStruct((NACC, 16), jnp.float32)]
        scratch += [pltpu.VMEM_SHARED((NACC, 16), jnp.float32)]

    @functools.partial(pl.kernel, out_type=out_type, mesh=_mesh,
                       scratch_types=scratch,
                       compiler_params=pltpu.CompilerParams(
                           use_tc_tiling_on_sc=False))
    def run(hub, hib, srcu, dstu, srci, dsti, zeros, zeros16, ones16, *rest):
        if do_cnt:
            (Si, Su, cnt_i, cnt_u, idxb, idxd,
             gb0, gb1, acc, ones_v, sg0, sg1, ss0, ss1, cacc) = rest
        else:
            (Si, Su, idxb, idxd,
             gb0, gb1, acc, ones_v, sg0, sg1, ss0, ss1) = rest
            cnt_i = cnt_u = cacc = None
        bufs = ((gb0, sg0, ss0), (gb1, sg1, ss1))
        s = lax.axis_index("s")
        c = lax.axis_index("c")

        def body(h_hbm, src_hbm, dst_hbm, out_hbm, cnt_hbm):
            pltpu.sync_copy(src_hbm.at[pl.ds(s * NCH, NCH)], idxb)
            pltpu.sync_copy(dst_hbm.at[pl.ds(s * NCH, NCH)], idxd)
            pltpu.sync_copy(zeros.at[pl.ds(s * RPT, RPT)],
                            acc.at[pl.ds(s * RPT, RPT)])
            if do_cnt:
                pltpu.sync_copy(ones16, ones_v)
                pltpu.sync_copy(zeros16.at[pl.ds(s * RPT, RPT)],
                                cacc.at[pl.ds(s * RPT, RPT)])
            plsc.subcore_barrier()

            for b, (gb, sg, ss) in enumerate(bufs):
                pltpu.async_copy(h_hbm.at[idxb.at[b]], gb, sg)

            @pl.loop(0, NCH, step=2)
            def _(t):
                for b, (gb, sg, ss) in enumerate(bufs):
                    tt = t + b
                    pltpu.make_async_copy(h_hbm.at[idxb.at[tt]],
                                          gb, sg).wait()
                    cp = pltpu.async_copy(gb, acc.at[idxd.at[tt]],
                                          ss, add=True)
                    if do_cnt:
                        pltpu.sync_copy(ones_v, cacc.at[idxd.at[tt]],
                                        add=True)
                    cp.wait()

                    @pl.when(tt + 2 < NCH)
                    def _():
                        pltpu.async_copy(h_hbm.at[idxb.at[tt + 2]], gb, sg)

            plsc.subcore_barrier()
            pltpu.sync_copy(acc.at[pl.ds(s * RPT, RPT)],
                            out_hbm.at[pl.ds(s * RPT, RPT)])
            if do_cnt:
                pltpu.sync_copy(cacc.at[pl.ds(s * RPT, RPT)],
                                cnt_hbm.at[pl.ds(s * RPT, RPT)])

        @pl.when(c == 0)
        def _():
            body(hub, srcu, dstu, Si, cnt_i)

        @pl.when(c == 1)
        def _():
            body(hib, srci, dsti, Su, cnt_u)

    return run(hub, hib, srcu, dstu, srci, dsti, zeros, zeros16, ones16)


def _tc_layer(final, Su, Si, hu, hi, cu, ci,
              Wl_u2i, bl_u2i, Wr_u2i, Wl_i2u, bl_i2u, Wr_i2u,
              Wlin_u=None, blin_u=None, Wlin_i=None, blin_i=None):
    odim = OUT if final else D

    def body(Su_r, Si_r, hu_r, hi_r, cu_r, ci_r,
             wlu, blu, wru, wli, bli, wri, *rest):
        inv_i = 1.0 / jnp.maximum(ci_r[:, 0:1], 1.0)
        inv_u = 1.0 / jnp.maximum(cu_r[:, 0:1], 1.0)
        agg_i = (Si_r[...].astype(jnp.float32) * inv_i) @ wlu[...]
        agg_u = (Su_r[...].astype(jnp.float32) * inv_u) @ wli[...]
        ni = jnp.maximum(agg_i + blu[...] + hi_r[...] @ wru[...], 0.0)
        nu = jnp.maximum(agg_u + bli[...] + hu_r[...] @ wri[...], 0.0)
        if final:
            wlinu, blinu, wlini, blini, out_u, out_i = rest
            out_u[...] = nu @ wlinu[...] + blinu[...]
            out_i[...] = ni @ wlini[...] + blini[...]
        else:
            out_u, out_i, out_ub, out_ib = rest
            out_u[...] = nu
            out_i[...] = ni
            out_ub[...] = nu.astype(jnp.bfloat16)
            out_ib[...] = ni.astype(jnp.bfloat16)

    args = [Su, Si, hu, hi, cu, ci,
            Wl_u2i, bl_u2i, Wr_u2i, Wl_i2u, bl_i2u, Wr_i2u]
    if final:
        args += [Wlin_u, blin_u, Wlin_i, blin_i]
    out_shape = [jax.ShapeDtypeStruct((N, odim), jnp.float32),
                 jax.ShapeDtypeStruct((N, odim), jnp.float32)]
    if not final:
        out_shape += [jax.ShapeDtypeStruct((N, D), jnp.bfloat16),
                      jax.ShapeDtypeStruct((N, D), jnp.bfloat16)]
    BRW = 1000
    blocked = lambda cols: pl.BlockSpec((BRW, cols), lambda i: (i, 0))
    full = lambda a: pl.BlockSpec(a.shape, lambda i: (0, 0))
    in_specs = [blocked(D)] * 4 + [blocked(16)] * 2
    in_specs += [full(a) for a in args[6:]]
    out_specs = [blocked(odim)] * 2
    if not final:
        out_specs += [blocked(D)] * 2
    return pl.pallas_call(
        body,
        grid=(N // BRW,),
        in_specs=in_specs,
        out_specs=out_specs,
        out_shape=out_shape,
    )(*args)


def _tc_decoder(z_u, z_i):
    BR = 200

    def body(zu_r, zi_r, out_r):
        logits = lax.dot_general(zu_r[...], zi_r[...],
                                 (((1,), (1,)), ((), ())),
                                 preferred_element_type=jnp.float32)
        out_r[...] = 1.0 / (1.0 + jnp.exp(-logits))

    return pl.pallas_call(
        body,
        grid=(N // BR,),
        in_specs=[pl.BlockSpec((BR, OUT), lambda i: (i, 0)),
                  pl.BlockSpec((N, OUT), lambda i: (0, 0))],
        out_specs=pl.BlockSpec((BR, N), lambda i: (i, 0)),
        out_shape=jax.ShapeDtypeStruct((N, N), jnp.float32),
    )(z_u, z_i)


def _prep_edges(ei):
    src = ei[0].astype(jnp.int32)
    dst = ei[1].astype(jnp.int32)
    pad = EPAD - E
    src = jnp.concatenate([src, jnp.zeros((pad,), jnp.int32)])
    dst = jnp.concatenate([dst, jnp.full((pad,), N, jnp.int32)])
    shape = (NSUB * NCH, CHUNK)
    return src.reshape(shape), dst.reshape(shape)


def kernel(x_user, x_item, params, edge_index_u2i, edge_index_i2u):
    srcu, dstu = _prep_edges(edge_index_u2i)
    srci, dsti = _prep_edges(edge_index_i2u)
    zeros = jnp.zeros((NACC, D), jnp.bfloat16)
    zeros16 = jnp.zeros((NACC, 16), jnp.float32)
    ones16 = jnp.ones((CHUNK, 16), jnp.float32)

    p = params
    b2 = lambda v: v.reshape(1, -1)

    hu, hi = x_user, x_item
    hub = x_user.astype(jnp.bfloat16)
    hib = x_item.astype(jnp.bfloat16)
    cu = ci = None
    for L in range(3):
        res = _sc_agg(L == 0, hub, hib, srcu, dstu, srci, dsti,
                      zeros, zeros16, ones16)
        if L == 0:
            Si, Su, ci_f, cu_f = res
            ci = ci_f[:N]
            cu = cu_f[:N]
        else:
            Si, Su = res
        final = L == 2
        extra = {}
        if final:
            extra = dict(Wlin_u=p['Wlin_user'], blin_u=b2(p['blin_user']),
                         Wlin_i=p['Wlin_item'], blin_i=b2(p['blin_item']))
        out = _tc_layer(final, Su[:N], Si[:N], hu, hi, cu, ci,
                        p['Wl%d_u2i' % L], b2(p['bl%d_u2i' % L]),
                        p['Wr%d_u2i' % L],
                        p['Wl%d_i2u' % L], b2(p['bl%d_i2u' % L]),
                        p['Wr%d_i2u' % L], **extra)
        if final:
            hu, hi = out
        else:
            hu, hi, hub, hib = out
    return _tc_decoder(hu, hi)
